# baseline (speedup 1.0000x reference)
import functools
import jax
import jax.numpy as jnp
from jax import lax
from jax.experimental import pallas as pl
from jax.experimental.pallas import tpu as pltpu
from jax.experimental.pallas import tpu_sc as plsc

N_NODES = 100
FEAT = 32
HIDDEN = 256
EMB = 64
GNN_HID = 256
GNN_OUT = 128
FLAT_DIM = 128
FLAT_OUT = 64
T = 200
E = 3200

N_A = 52
N_B = N_NODES - N_A

N_TILES = 32
E_TILE = 112
E_PAD = N_TILES * E_TILE
M_COLS = 104
M_FLAT = N_NODES * M_COLS

_NT = (((1,), (1,)), ((), ()))


def _sc_edges_body(src_hbm, dst_hbm, val_hbm, zero_hbm, out_hbm,
                   srcv, dstv, valv, fidxv, sh_a, sh_b):
    c = lax.axis_index("c")
    s = lax.axis_index("s")
    wid = s * 2 + c
    base = wid * E_TILE
    pltpu.sync_copy(src_hbm.at[pl.ds(base, E_TILE)], srcv)
    pltpu.sync_copy(dst_hbm.at[pl.ds(base, E_TILE)], dstv)
    pltpu.sync_copy(val_hbm.at[pl.ds(base, E_TILE)], valv)

    @pl.when(jnp.logical_and(c == 0, s == 0))
    def _():
        pltpu.sync_copy(zero_hbm, sh_a)

    @pl.when(jnp.logical_and(c == 1, s == 0))
    def _():
        pltpu.sync_copy(zero_hbm, sh_b)

    for j in range(E_TILE // 16):
        d = dstv[pl.ds(j * 16, 16)]
        sv = srcv[pl.ds(j * 16, 16)]
        fidxv[pl.ds(j * 16, 16)] = d * M_COLS + sv

    plsc.subcore_barrier()

    @pl.when(c == 0)
    def _():
        pltpu.sync_copy(valv, sh_a.at[fidxv], add=True)

    @pl.when(c == 1)
    def _():
        pltpu.sync_copy(valv, sh_b.at[fidxv], add=True)

    plsc.subcore_barrier()

    @pl.when(jnp.logical_and(c == 0, s == 0))
    def _():
        pltpu.sync_copy(sh_a, out_hbm.at[0])

    @pl.when(jnp.logical_and(c == 1, s == 0))
    def _():
        pltpu.sync_copy(sh_b, out_hbm.at[1])


def _sc_edge_partials(src, dst, val, zero):
    mesh = plsc.VectorSubcoreMesh(core_axis_name="c", subcore_axis_name="s")
    k = functools.partial(
        pl.kernel, mesh=mesh,
        out_type=jax.ShapeDtypeStruct((2, M_FLAT), jnp.float32),
        scratch_types=[
            pltpu.VMEM((E_TILE,), jnp.int32),
            pltpu.VMEM((E_TILE,), jnp.int32),
            pltpu.VMEM((E_TILE,), jnp.float32),
            pltpu.VMEM((E_TILE,), jnp.int32),
            pltpu.VMEM_SHARED((M_FLAT,), jnp.float32),
            pltpu.VMEM_SHARED((M_FLAT,), jnp.float32),
        ],
    )(_sc_edges_body)
    return k(src, dst, val, zero)


def _gru_body(nf_ref, wihT_ref, whhT_ref, bias_ref, hout_ref):
    f32 = jnp.float32
    bf16 = jnp.bfloat16

    wihT = wihT_ref[...]
    whhT = whhT_ref[...]
    bias = bias_ref[...]

    def gates(gi, gh, h):
        r = jnp.tanh((gi[:, :HIDDEN] + gh[:, :HIDDEN]) * 0.5) * 0.5 + 0.5
        z = jnp.tanh((gi[:, HIDDEN:2 * HIDDEN] + gh[:, HIDDEN:2 * HIDDEN]) * 0.5) * 0.5 + 0.5
        n = jnp.tanh(gi[:, 2 * HIDDEN:] + r * gh[:, 2 * HIDDEN:])
        return n + z * (h - n)

    def substep(t, ha, hb):
        x_t = nf_ref[t]
        gia = jnp.dot(x_t[:N_A], wihT, preferred_element_type=f32) + bias
        gha = jnp.dot(ha.astype(bf16), whhT, preferred_element_type=f32)
        gib = jnp.dot(x_t[N_A:], wihT, preferred_element_type=f32) + bias
        ghb = jnp.dot(hb.astype(bf16), whhT, preferred_element_type=f32)
        return gates(gia, gha, ha), gates(gib, ghb, hb)

    def step(i, carry):
        ha, hb = carry
        t = i * 2
        ha, hb = substep(t, ha, hb)
        ha, hb = substep(t + 1, ha, hb)
        return ha, hb

    ha, hb = jax.lax.fori_loop(
        0, T // 2, step,
        (jnp.zeros((N_A, HIDDEN), f32), jnp.zeros((N_B, HIDDEN), f32)))
    hout_ref[...] = jnp.concatenate([ha, hb], axis=0)


def _gnn_body(h_ref, flat_ref, m_ref, emb_ref, ws1_ref, wn1_ref, b1_ref,
              ws2_ref, wn2_ref, b2_ref, fw_ref, fb_ref, ow_ref, ob_ref,
              out_ref):
    f32 = jnp.float32
    h_last = h_ref[...]

    mp = m_ref[...]
    m = (mp[0] + mp[1])[:, :N_NODES]
    cnt = jnp.sum(m, axis=1, keepdims=True)
    inv_cnt = 1.0 / jnp.maximum(cnt, 1.0)

    emb = emb_ref[...]
    gnn_in = jnp.concatenate([h_last, emb], axis=1)
    mean1 = jnp.dot(m, gnn_in, preferred_element_type=f32) * inv_cnt
    h1 = jnp.dot(gnn_in, ws1_ref[...], preferred_element_type=f32)
    h1 = h1 + jnp.dot(mean1, wn1_ref[...], preferred_element_type=f32)
    h1 = jax.nn.relu(h1 + b1_ref[...])

    mean2 = jnp.dot(m, h1, preferred_element_type=f32) * inv_cnt
    h2 = jnp.dot(h1, ws2_ref[...], preferred_element_type=f32)
    h2 = h2 + jnp.dot(mean2, wn2_ref[...], preferred_element_type=f32)
    h2 = h2 + b2_ref[...]

    xflat = jnp.dot(flat_ref[...], fw_ref[...], preferred_element_type=f32) + fb_ref[...]
    xcat = jnp.concatenate([h2, xflat, h_last], axis=1)
    out = jnp.dot(xcat, ow_ref[...], preferred_element_type=f32) + ob_ref[...]
    out_ref[...] = out


def kernel(node_feat, flat, edge_index, W_ih, W_hh, b_ih, b_hh, emb_weight,
           W_self1, W_neigh1, b1, W_self2, W_neigh2, b2, flat_W, flat_b,
           out_W, out_b):
    f32 = jnp.float32
    bf16 = jnp.bfloat16
    nf = node_feat.astype(bf16)

    epad = ((0, E_PAD - E),)
    src = jnp.pad(edge_index[0], epad)
    dst = jnp.pad(edge_index[1], epad)
    val = (jnp.arange(E_PAD) < E).astype(f32)
    zero = jnp.zeros((M_FLAT,), f32)
    m_part = _sc_edge_partials(src, dst, val, zero)
    m3d = m_part.reshape(2, N_NODES, M_COLS)

    h_last = pl.pallas_call(
        _gru_body,
        out_shape=jax.ShapeDtypeStruct((N_NODES, HIDDEN), f32),
    )(
        nf,
        W_ih.T.astype(bf16), W_hh.T.astype(bf16),
        (b_ih + b_hh).reshape(1, -1),
    )

    out = pl.pallas_call(
        _gnn_body,
        out_shape=jax.ShapeDtypeStruct((N_NODES, 1), f32),
    )(
        h_last, flat, m3d,
        emb_weight, W_self1, W_neigh1, b1.reshape(1, -1),
        W_self2, W_neigh2, b2.reshape(1, -1),
        flat_W, flat_b.reshape(1, -1), out_W, out_b.reshape(1, -1),
    )
    return out[:, 0]


# speedup vs baseline: 1.0285x; 1.0285x over previous
import functools
import jax
import jax.numpy as jnp
from jax import lax
from jax.experimental import pallas as pl
from jax.experimental.pallas import tpu as pltpu
from jax.experimental.pallas import tpu_sc as plsc

N_NODES = 100
FEAT = 32
HIDDEN = 256
EMB = 64
GNN_HID = 256
GNN_OUT = 128
FLAT_DIM = 128
FLAT_OUT = 64
T = 200
E = 3200

N_A = 52
N_B = N_NODES - N_A

N_TILES = 32
E_TILE = 112
E_PAD = N_TILES * E_TILE
M_COLS = 104
M_FLAT = N_NODES * M_COLS

_NT = (((1,), (1,)), ((), ()))


def _sc_edges_body(src_hbm, dst_hbm, zero_hbm, out_hbm,
                   srcv, dstv, valv, fidxv, sh_a, sh_b):
    c = lax.axis_index("c")
    s = lax.axis_index("s")
    wid = s * 2 + c
    base = wid * E_TILE
    pltpu.sync_copy(src_hbm.at[pl.ds(base, E_TILE)], srcv)
    pltpu.sync_copy(dst_hbm.at[pl.ds(base, E_TILE)], dstv)

    @pl.when(jnp.logical_and(c == 0, s == 0))
    def _():
        pltpu.sync_copy(zero_hbm, sh_a)

    @pl.when(jnp.logical_and(c == 1, s == 0))
    def _():
        pltpu.sync_copy(zero_hbm, sh_b)

    for j in range(E_TILE // 16):
        d = dstv[pl.ds(j * 16, 16)]
        sv = srcv[pl.ds(j * 16, 16)]
        fidxv[pl.ds(j * 16, 16)] = d * M_COLS + sv
        valv[pl.ds(j * 16, 16)] = jnp.full((16,), 1.0, jnp.float32)

    plsc.subcore_barrier()

    @pl.when(c == 0)
    def _():
        pltpu.sync_copy(valv, sh_a.at[fidxv], add=True)

    @pl.when(c == 1)
    def _():
        pltpu.sync_copy(valv, sh_b.at[fidxv], add=True)

    plsc.subcore_barrier()

    @pl.when(jnp.logical_and(c == 0, s == 0))
    def _():
        pltpu.sync_copy(sh_a, out_hbm.at[0])

    @pl.when(jnp.logical_and(c == 1, s == 0))
    def _():
        pltpu.sync_copy(sh_b, out_hbm.at[1])


def _sc_edge_partials(src, dst, zero):
    mesh = plsc.VectorSubcoreMesh(core_axis_name="c", subcore_axis_name="s")
    k = functools.partial(
        pl.kernel, mesh=mesh,
        out_type=jax.ShapeDtypeStruct((2, M_FLAT), jnp.float32),
        scratch_types=[
            pltpu.VMEM((E_TILE,), jnp.int32),
            pltpu.VMEM((E_TILE,), jnp.int32),
            pltpu.VMEM((E_TILE,), jnp.float32),
            pltpu.VMEM((E_TILE,), jnp.int32),
            pltpu.VMEM_SHARED((M_FLAT,), jnp.float32),
            pltpu.VMEM_SHARED((M_FLAT,), jnp.float32),
        ],
    )(_sc_edges_body)
    return k(src, dst, zero)


def _fused_body(nf_ref, flat_ref, m_ref, wihT_ref,
                whhT_ref, bias_ref, emb_ref, ws1_ref, wn1_ref, b1_ref,
                ws2_ref, wn2_ref, b2_ref, fw_ref, fb_ref, ow_ref, ob_ref,
                out_ref):
    f32 = jnp.float32
    bf16 = jnp.bfloat16

    wihT = wihT_ref[...]
    whhT = whhT_ref[...]
    bias = bias_ref[...]

    def gates(gi, gh, h):
        r = jnp.tanh((gi[:, :HIDDEN] + gh[:, :HIDDEN]) * 0.5) * 0.5 + 0.5
        z = jnp.tanh((gi[:, HIDDEN:2 * HIDDEN] + gh[:, HIDDEN:2 * HIDDEN]) * 0.5) * 0.5 + 0.5
        n = jnp.tanh(gi[:, 2 * HIDDEN:] + r * gh[:, 2 * HIDDEN:])
        return n + z * (h - n)

    def substep(t, ha, hb):
        x_t = nf_ref[t]
        gia = jnp.dot(x_t[:N_A], wihT, preferred_element_type=f32) + bias
        gha = jnp.dot(ha.astype(bf16), whhT, preferred_element_type=f32)
        gib = jnp.dot(x_t[N_A:], wihT, preferred_element_type=f32) + bias
        ghb = jnp.dot(hb.astype(bf16), whhT, preferred_element_type=f32)
        return gates(gia, gha, ha), gates(gib, ghb, hb)

    def step(i, carry):
        ha, hb = carry
        t = i * 2
        ha, hb = substep(t, ha, hb)
        ha, hb = substep(t + 1, ha, hb)
        return ha, hb

    ha, hb = jax.lax.fori_loop(
        0, T // 2, step,
        (jnp.zeros((N_A, HIDDEN), f32), jnp.zeros((N_B, HIDDEN), f32)))
    h_last = jnp.concatenate([ha, hb], axis=0)

    m = m_ref[...][:, :N_NODES]
    cnt = jnp.sum(m, axis=1, keepdims=True)
    inv_cnt = 1.0 / jnp.maximum(cnt, 1.0)

    emb = emb_ref[...]
    gnn_in = jnp.concatenate([h_last, emb], axis=1)
    mean1 = jnp.dot(m, gnn_in, preferred_element_type=f32) * inv_cnt
    h1 = jnp.dot(gnn_in, ws1_ref[...], preferred_element_type=f32)
    h1 = h1 + jnp.dot(mean1, wn1_ref[...], preferred_element_type=f32)
    h1 = jax.nn.relu(h1 + b1_ref[...])

    mean2 = jnp.dot(m, h1, preferred_element_type=f32) * inv_cnt
    h2 = jnp.dot(h1, ws2_ref[...], preferred_element_type=f32)
    h2 = h2 + jnp.dot(mean2, wn2_ref[...], preferred_element_type=f32)
    h2 = h2 + b2_ref[...]

    xflat = jnp.dot(flat_ref[...], fw_ref[...], preferred_element_type=f32) + fb_ref[...]
    xcat = jnp.concatenate([h2, xflat, h_last], axis=1)
    out = jnp.dot(xcat, ow_ref[...], preferred_element_type=f32) + ob_ref[...]
    out_ref[...] = out


def kernel(node_feat, flat, edge_index, W_ih, W_hh, b_ih, b_hh, emb_weight,
           W_self1, W_neigh1, b1, W_self2, W_neigh2, b2, flat_W, flat_b,
           out_W, out_b):
    f32 = jnp.float32
    bf16 = jnp.bfloat16
    nf = node_feat.astype(bf16)

    epad = ((0, E_PAD - E),)
    src = jnp.pad(edge_index[0], epad, constant_values=N_NODES)
    dst = jnp.pad(edge_index[1], epad)
    zero = jnp.zeros((M_FLAT,), f32)
    m_part = _sc_edge_partials(src, dst, zero)
    m2d = m_part.sum(axis=0).reshape(N_NODES, M_COLS)

    out = pl.pallas_call(
        _fused_body,
        out_shape=jax.ShapeDtypeStruct((N_NODES, 1), f32),
    )(
        nf, flat, m2d,
        W_ih.T.astype(bf16), W_hh.T.astype(bf16),
        (b_ih + b_hh).reshape(1, -1),
        emb_weight, W_self1, W_neigh1, b1.reshape(1, -1),
        W_self2, W_neigh2, b2.reshape(1, -1),
        flat_W, flat_b.reshape(1, -1), out_W, out_b.reshape(1, -1),
    )
    return out[:, 0]


# speedup vs baseline: 1.0292x; 1.0007x over previous
import functools
import jax
import jax.numpy as jnp
from jax import lax
from jax.experimental import pallas as pl
from jax.experimental.pallas import tpu as pltpu
from jax.experimental.pallas import tpu_sc as plsc

N_NODES = 100
FEAT = 32
HIDDEN = 256
EMB = 64
GNN_HID = 256
GNN_OUT = 128
FLAT_DIM = 128
FLAT_OUT = 64
T = 200
E = 3200

N_A = 52
N_B = N_NODES - N_A

N_TILES = 32
E_TILE = 112
E_PAD = N_TILES * E_TILE
M_COLS = 104
M_FLAT = N_NODES * M_COLS

_NT = (((1,), (1,)), ((), ()))


def _sc_edges_body(src_hbm, dst_hbm, zero_hbm, out_hbm,
                   srcv, dstv, valv, fidxv, sh_a, sh_b):
    c = lax.axis_index("c")
    s = lax.axis_index("s")
    wid = s * 2 + c
    base = wid * E_TILE
    pltpu.sync_copy(src_hbm.at[pl.ds(base, E_TILE)], srcv)
    pltpu.sync_copy(dst_hbm.at[pl.ds(base, E_TILE)], dstv)

    @pl.when(jnp.logical_and(c == 0, s == 0))
    def _():
        pltpu.sync_copy(zero_hbm, sh_a)

    @pl.when(jnp.logical_and(c == 1, s == 0))
    def _():
        pltpu.sync_copy(zero_hbm, sh_b)

    for j in range(E_TILE // 16):
        d = dstv[pl.ds(j * 16, 16)]
        sv = srcv[pl.ds(j * 16, 16)]
        fidxv[pl.ds(j * 16, 16)] = d * M_COLS + sv
        valv[pl.ds(j * 16, 16)] = jnp.full((16,), 1.0, jnp.float32)

    plsc.subcore_barrier()

    @pl.when(c == 0)
    def _():
        pltpu.sync_copy(valv, sh_a.at[fidxv], add=True)

    @pl.when(c == 1)
    def _():
        pltpu.sync_copy(valv, sh_b.at[fidxv], add=True)

    plsc.subcore_barrier()

    @pl.when(jnp.logical_and(c == 0, s == 0))
    def _():
        pltpu.sync_copy(sh_a, out_hbm.at[0])

    @pl.when(jnp.logical_and(c == 1, s == 0))
    def _():
        pltpu.sync_copy(sh_b, out_hbm.at[1])


def _sc_edge_partials(src, dst, zero):
    mesh = plsc.VectorSubcoreMesh(core_axis_name="c", subcore_axis_name="s")
    k = functools.partial(
        pl.kernel, mesh=mesh,
        out_type=jax.ShapeDtypeStruct((2, M_FLAT), jnp.float32),
        scratch_types=[
            pltpu.VMEM((E_TILE,), jnp.int32),
            pltpu.VMEM((E_TILE,), jnp.int32),
            pltpu.VMEM((E_TILE,), jnp.float32),
            pltpu.VMEM((E_TILE,), jnp.int32),
            pltpu.VMEM_SHARED((M_FLAT,), jnp.float32),
            pltpu.VMEM_SHARED((M_FLAT,), jnp.float32),
        ],
    )(_sc_edges_body)
    return k(src, dst, zero)


def _fused_body(nf_ref, flat_ref, m_ref, wihT_ref,
                whhT_ref, bias_ref, emb_ref, ws1_ref, wn1_ref, b1_ref,
                ws2_ref, wn2_ref, b2_ref, fw_ref, fb_ref, ow_ref, ob_ref,
                out_ref):
    f32 = jnp.float32
    bf16 = jnp.bfloat16

    wihT = wihT_ref[...]
    whhT = whhT_ref[...]
    bias = bias_ref[...]

    def gates(gi, gh, h):
        r = jnp.tanh((gi[:, :HIDDEN] + gh[:, :HIDDEN]) * 0.5) * 0.5 + 0.5
        z = jnp.tanh((gi[:, HIDDEN:2 * HIDDEN] + gh[:, HIDDEN:2 * HIDDEN]) * 0.5) * 0.5 + 0.5
        n = jnp.tanh(gi[:, 2 * HIDDEN:] + r * gh[:, 2 * HIDDEN:])
        return n + z * (h - n)

    def substep(t, ha, hb):
        x_t = nf_ref[t]
        gia = jnp.dot(x_t[:N_A], wihT, preferred_element_type=f32) + bias
        gha = jnp.dot(ha.astype(bf16), whhT, preferred_element_type=f32)
        gib = jnp.dot(x_t[N_A:], wihT, preferred_element_type=f32) + bias
        ghb = jnp.dot(hb.astype(bf16), whhT, preferred_element_type=f32)
        return gates(gia, gha, ha), gates(gib, ghb, hb)

    def step(i, carry):
        ha, hb = carry
        t = i * 2
        ha, hb = substep(t, ha, hb)
        ha, hb = substep(t + 1, ha, hb)
        return ha, hb

    ha, hb = jax.lax.fori_loop(
        0, T // 2, step,
        (jnp.zeros((N_A, HIDDEN), f32), jnp.zeros((N_B, HIDDEN), f32)))
    h_last = jnp.concatenate([ha, hb], axis=0)

    m = (m_ref[0] + m_ref[1])[:, :N_NODES]
    cnt = jnp.sum(m, axis=1, keepdims=True)
    inv_cnt = 1.0 / jnp.maximum(cnt, 1.0)

    emb = emb_ref[...]
    gnn_in = jnp.concatenate([h_last, emb], axis=1)
    mean1 = jnp.dot(m, gnn_in, preferred_element_type=f32) * inv_cnt
    h1 = jnp.dot(gnn_in, ws1_ref[...], preferred_element_type=f32)
    h1 = h1 + jnp.dot(mean1, wn1_ref[...], preferred_element_type=f32)
    h1 = jax.nn.relu(h1 + b1_ref[...])

    mean2 = jnp.dot(m, h1, preferred_element_type=f32) * inv_cnt
    h2 = jnp.dot(h1, ws2_ref[...], preferred_element_type=f32)
    h2 = h2 + jnp.dot(mean2, wn2_ref[...], preferred_element_type=f32)
    h2 = h2 + b2_ref[...]

    xflat = jnp.dot(flat_ref[...], fw_ref[...], preferred_element_type=f32) + fb_ref[...]
    xcat = jnp.concatenate([h2, xflat, h_last], axis=1)
    out = jnp.dot(xcat, ow_ref[...], preferred_element_type=f32) + ob_ref[...]
    out_ref[...] = out


def kernel(node_feat, flat, edge_index, W_ih, W_hh, b_ih, b_hh, emb_weight,
           W_self1, W_neigh1, b1, W_self2, W_neigh2, b2, flat_W, flat_b,
           out_W, out_b):
    f32 = jnp.float32
    bf16 = jnp.bfloat16
    nf = node_feat.astype(bf16)

    epad = ((0, E_PAD - E),)
    src = jnp.pad(edge_index[0], epad, constant_values=N_NODES)
    dst = jnp.pad(edge_index[1], epad)
    zero = jnp.zeros((M_FLAT,), f32)
    m_part = _sc_edge_partials(src, dst, zero)
    m3d = m_part.reshape(2, N_NODES, M_COLS)

    out = pl.pallas_call(
        _fused_body,
        out_shape=jax.ShapeDtypeStruct((N_NODES, 1), f32),
    )(
        nf, flat, m3d,
        W_ih.T.astype(bf16), W_hh.T.astype(bf16),
        (b_ih + b_hh).reshape(1, -1),
        emb_weight, W_self1, W_neigh1, b1.reshape(1, -1),
        W_self2, W_neigh2, b2.reshape(1, -1),
        flat_W, flat_b.reshape(1, -1), out_W, out_b.reshape(1, -1),
    )
    return out[:, 0]


# speedup vs baseline: 1.0939x; 1.0628x over previous
import functools
import jax
import jax.numpy as jnp
from jax import lax
from jax.experimental import pallas as pl
from jax.experimental.pallas import tpu as pltpu
from jax.experimental.pallas import tpu_sc as plsc

N_NODES = 100
FEAT = 32
HIDDEN = 256
EMB = 64
GNN_HID = 256
GNN_OUT = 128
FLAT_DIM = 128
FLAT_OUT = 64
T = 200
E = 3200

N_A = 52
N_B = N_NODES - N_A

N_TILES = 32
E_TILE = 112
E_PAD = N_TILES * E_TILE
M_COLS = 104
M_FLAT = N_NODES * M_COLS


def _sc_edges_body(src_hbm, dst_hbm, zero_hbm, out_hbm,
                   srcv, dstv, valv, fidxv, sh_a, sh_b):
    c = lax.axis_index("c")
    s = lax.axis_index("s")
    wid = s * 2 + c
    base = wid * E_TILE
    pltpu.sync_copy(src_hbm.at[pl.ds(base, E_TILE)], srcv)
    pltpu.sync_copy(dst_hbm.at[pl.ds(base, E_TILE)], dstv)

    @pl.when(jnp.logical_and(c == 0, s == 0))
    def _():
        pltpu.sync_copy(zero_hbm, sh_a)

    @pl.when(jnp.logical_and(c == 1, s == 0))
    def _():
        pltpu.sync_copy(zero_hbm, sh_b)

    for j in range(E_TILE // 16):
        d = dstv[pl.ds(j * 16, 16)]
        sv = srcv[pl.ds(j * 16, 16)]
        fidxv[pl.ds(j * 16, 16)] = d * M_COLS + sv
        valv[pl.ds(j * 16, 16)] = jnp.full((16,), 1.0, jnp.float32)

    plsc.subcore_barrier()

    @pl.when(c == 0)
    def _():
        pltpu.sync_copy(valv, sh_a.at[fidxv], add=True)

    @pl.when(c == 1)
    def _():
        pltpu.sync_copy(valv, sh_b.at[fidxv], add=True)

    plsc.subcore_barrier()

    @pl.when(jnp.logical_and(c == 0, s == 0))
    def _():
        pltpu.sync_copy(sh_a, out_hbm.at[0])

    @pl.when(jnp.logical_and(c == 1, s == 0))
    def _():
        pltpu.sync_copy(sh_b, out_hbm.at[1])


def _sc_edge_partials(src, dst, zero):
    mesh = plsc.VectorSubcoreMesh(core_axis_name="c", subcore_axis_name="s")
    k = functools.partial(
        pl.kernel, mesh=mesh,
        out_type=jax.ShapeDtypeStruct((2, M_FLAT), jnp.float32),
        scratch_types=[
            pltpu.VMEM((E_TILE,), jnp.int32),
            pltpu.VMEM((E_TILE,), jnp.int32),
            pltpu.VMEM((E_TILE,), jnp.float32),
            pltpu.VMEM((E_TILE,), jnp.int32),
            pltpu.VMEM_SHARED((M_FLAT,), jnp.float32),
            pltpu.VMEM_SHARED((M_FLAT,), jnp.float32),
        ],
    )(_sc_edges_body)
    return k(src, dst, zero)


def _fused_body(nf_ref, flat_ref, m_ref, wihT_ref,
                whhT_ref, bias_ref, emb_ref, ws1_ref, wn1_ref, b1_ref,
                ws2_ref, wn2_ref, b2_ref, fw_ref, fb_ref, ow_ref, ob_ref,
                out_ref):
    f32 = jnp.float32
    bf16 = jnp.bfloat16

    wihT = wihT_ref[...]
    whhT = whhT_ref[...]
    bias = bias_ref[...]

    def gates(gi, gh, h):
        r = jnp.tanh((gi[:, :HIDDEN] + gh[:, :HIDDEN]) * 0.5) * 0.5 + 0.5
        z = jnp.tanh((gi[:, HIDDEN:2 * HIDDEN] + gh[:, HIDDEN:2 * HIDDEN]) * 0.5) * 0.5 + 0.5
        n = jnp.tanh(gi[:, 2 * HIDDEN:] + r * gh[:, 2 * HIDDEN:])
        return n + z * (h - n)

    def substep(t, ha, hb):
        x_t = nf_ref[t]
        gia = jnp.dot(x_t[:N_A], wihT, preferred_element_type=f32) + bias
        gha = jnp.dot(ha.astype(bf16), whhT, preferred_element_type=f32)
        gib = jnp.dot(x_t[N_A:], wihT, preferred_element_type=f32) + bias
        ghb = jnp.dot(hb.astype(bf16), whhT, preferred_element_type=f32)
        return gates(gia, gha, ha), gates(gib, ghb, hb)

    def step(i, carry):
        ha, hb = carry
        t = i * 4
        ha, hb = substep(t, ha, hb)
        ha, hb = substep(t + 1, ha, hb)
        ha, hb = substep(t + 2, ha, hb)
        ha, hb = substep(t + 3, ha, hb)
        return ha, hb

    ha, hb = jax.lax.fori_loop(
        0, T // 4, step,
        (jnp.zeros((N_A, HIDDEN), f32), jnp.zeros((N_B, HIDDEN), f32)))
    h_last = jnp.concatenate([ha, hb], axis=0)

    m = (m_ref[0] + m_ref[1])[:, :N_NODES]
    cnt = jnp.sum(m, axis=1, keepdims=True)
    inv_cnt = 1.0 / jnp.maximum(cnt, 1.0)

    emb = emb_ref[...]
    gnn_in = jnp.concatenate([h_last, emb], axis=1)
    mean1 = jnp.dot(m, gnn_in, preferred_element_type=f32) * inv_cnt
    h1 = jnp.dot(gnn_in, ws1_ref[...], preferred_element_type=f32)
    h1 = h1 + jnp.dot(mean1, wn1_ref[...], preferred_element_type=f32)
    h1 = jax.nn.relu(h1 + b1_ref[...])

    mean2 = jnp.dot(m, h1, preferred_element_type=f32) * inv_cnt
    h2 = jnp.dot(h1, ws2_ref[...], preferred_element_type=f32)
    h2 = h2 + jnp.dot(mean2, wn2_ref[...], preferred_element_type=f32)
    h2 = h2 + b2_ref[...]

    xflat = jnp.dot(flat_ref[...], fw_ref[...], preferred_element_type=f32) + fb_ref[...]
    xcat = jnp.concatenate([h2, xflat, h_last], axis=1)
    out = jnp.dot(xcat, ow_ref[...], preferred_element_type=f32) + ob_ref[...]
    out_ref[...] = out


def kernel(node_feat, flat, edge_index, W_ih, W_hh, b_ih, b_hh, emb_weight,
           W_self1, W_neigh1, b1, W_self2, W_neigh2, b2, flat_W, flat_b,
           out_W, out_b):
    f32 = jnp.float32
    bf16 = jnp.bfloat16
    nf = node_feat.astype(bf16)

    epad = ((0, E_PAD - E),)
    src = jnp.pad(edge_index[0], epad, constant_values=N_NODES)
    dst = jnp.pad(edge_index[1], epad)
    zero = jnp.zeros((M_FLAT,), f32)
    m_part = _sc_edge_partials(src, dst, zero)
    m3d = m_part.reshape(2, N_NODES, M_COLS)

    out = pl.pallas_call(
        _fused_body,
        out_shape=jax.ShapeDtypeStruct((N_NODES, 1), f32),
    )(
        nf, flat, m3d,
        W_ih.T.astype(bf16), W_hh.T.astype(bf16),
        (b_ih + b_hh).reshape(1, -1),
        emb_weight, W_self1, W_neigh1, b1.reshape(1, -1),
        W_self2, W_neigh2, b2.reshape(1, -1),
        flat_W, flat_b.reshape(1, -1), out_W, out_b.reshape(1, -1),
    )
    return out[:, 0]


# speedup vs baseline: 1.1438x; 1.0456x over previous
import functools
import jax
import jax.numpy as jnp
from jax import lax
from jax.experimental import pallas as pl
from jax.experimental.pallas import tpu as pltpu
from jax.experimental.pallas import tpu_sc as plsc

N_NODES = 100
FEAT = 32
HIDDEN = 256
EMB = 64
GNN_HID = 256
GNN_OUT = 128
FLAT_DIM = 128
FLAT_OUT = 64
T = 200
E = 3200

N_A = 52
N_B = N_NODES - N_A

N_TILES = 32
E_TILE = 112
E_PAD = N_TILES * E_TILE
M_COLS = 104
M_FLAT = N_NODES * M_COLS


def _sc_edges_body(src_hbm, dst_hbm, zero_hbm, out_hbm,
                   srcv, dstv, valv, fidxv, sh_a, sh_b):
    c = lax.axis_index("c")
    s = lax.axis_index("s")
    wid = s * 2 + c
    base = wid * E_TILE
    pltpu.sync_copy(src_hbm.at[pl.ds(base, E_TILE)], srcv)
    pltpu.sync_copy(dst_hbm.at[pl.ds(base, E_TILE)], dstv)

    @pl.when(jnp.logical_and(c == 0, s == 0))
    def _():
        pltpu.sync_copy(zero_hbm, sh_a)

    @pl.when(jnp.logical_and(c == 1, s == 0))
    def _():
        pltpu.sync_copy(zero_hbm, sh_b)

    for j in range(E_TILE // 16):
        d = dstv[pl.ds(j * 16, 16)]
        sv = srcv[pl.ds(j * 16, 16)]
        fidxv[pl.ds(j * 16, 16)] = d * M_COLS + sv
        valv[pl.ds(j * 16, 16)] = jnp.full((16,), 1.0, jnp.float32)

    plsc.subcore_barrier()

    @pl.when(c == 0)
    def _():
        pltpu.sync_copy(valv, sh_a.at[fidxv], add=True)

    @pl.when(c == 1)
    def _():
        pltpu.sync_copy(valv, sh_b.at[fidxv], add=True)

    plsc.subcore_barrier()

    @pl.when(jnp.logical_and(c == 0, s == 0))
    def _():
        pltpu.sync_copy(sh_a, out_hbm.at[0])

    @pl.when(jnp.logical_and(c == 1, s == 0))
    def _():
        pltpu.sync_copy(sh_b, out_hbm.at[1])


def _sc_edge_partials(src, dst, zero):
    mesh = plsc.VectorSubcoreMesh(core_axis_name="c", subcore_axis_name="s")
    k = functools.partial(
        pl.kernel, mesh=mesh,
        out_type=jax.ShapeDtypeStruct((2, M_FLAT), jnp.float32),
        scratch_types=[
            pltpu.VMEM((E_TILE,), jnp.int32),
            pltpu.VMEM((E_TILE,), jnp.int32),
            pltpu.VMEM((E_TILE,), jnp.float32),
            pltpu.VMEM((E_TILE,), jnp.int32),
            pltpu.VMEM_SHARED((M_FLAT,), jnp.float32),
            pltpu.VMEM_SHARED((M_FLAT,), jnp.float32),
        ],
    )(_sc_edges_body)
    return k(src, dst, zero)


def _fused_body(nf_ref, flat_ref, m_ref, wihT_ref,
                whhT_ref, bias_ref, emb_ref, ws1_ref, wn1_ref, b1_ref,
                ws2_ref, wn2_ref, b2_ref, fw_ref, fb_ref, ow_ref, ob_ref,
                out_ref):
    f32 = jnp.float32
    bf16 = jnp.bfloat16

    wihT = wihT_ref[...]
    whhT = whhT_ref[...]
    bias = bias_ref[...]

    def gates(gi, gh, h):
        r = jnp.tanh((gi[:, :HIDDEN] + gh[:, :HIDDEN]) * 0.5) * 0.5 + 0.5
        z = jnp.tanh((gi[:, HIDDEN:2 * HIDDEN] + gh[:, HIDDEN:2 * HIDDEN]) * 0.5) * 0.5 + 0.5
        n = jnp.tanh(gi[:, 2 * HIDDEN:] + r * gh[:, 2 * HIDDEN:])
        return n + z * (h - n)

    def substep(t, ha, hb):
        x_t = nf_ref[t]
        gia = jnp.dot(x_t[:N_A], wihT, preferred_element_type=f32) + bias
        gha = jnp.dot(ha.astype(bf16), whhT, preferred_element_type=f32)
        gib = jnp.dot(x_t[N_A:], wihT, preferred_element_type=f32) + bias
        ghb = jnp.dot(hb.astype(bf16), whhT, preferred_element_type=f32)
        return gates(gia, gha, ha), gates(gib, ghb, hb)

    def step(i, carry):
        ha, hb = carry
        t = i * 8
        for u in range(8):
            ha, hb = substep(t + u, ha, hb)
        return ha, hb

    ha, hb = jax.lax.fori_loop(
        0, T // 8, step,
        (jnp.zeros((N_A, HIDDEN), f32), jnp.zeros((N_B, HIDDEN), f32)))
    h_last = jnp.concatenate([ha, hb], axis=0)

    m = (m_ref[0] + m_ref[1])[:, :N_NODES]
    cnt = jnp.sum(m, axis=1, keepdims=True)
    inv_cnt = 1.0 / jnp.maximum(cnt, 1.0)

    emb = emb_ref[...]
    gnn_in = jnp.concatenate([h_last, emb], axis=1)
    mean1 = jnp.dot(m, gnn_in, preferred_element_type=f32) * inv_cnt
    h1 = jnp.dot(gnn_in, ws1_ref[...], preferred_element_type=f32)
    h1 = h1 + jnp.dot(mean1, wn1_ref[...], preferred_element_type=f32)
    h1 = jax.nn.relu(h1 + b1_ref[...])

    mean2 = jnp.dot(m, h1, preferred_element_type=f32) * inv_cnt
    h2 = jnp.dot(h1, ws2_ref[...], preferred_element_type=f32)
    h2 = h2 + jnp.dot(mean2, wn2_ref[...], preferred_element_type=f32)
    h2 = h2 + b2_ref[...]

    xflat = jnp.dot(flat_ref[...], fw_ref[...], preferred_element_type=f32) + fb_ref[...]
    xcat = jnp.concatenate([h2, xflat, h_last], axis=1)
    out = jnp.dot(xcat, ow_ref[...], preferred_element_type=f32) + ob_ref[...]
    out_ref[...] = out


def kernel(node_feat, flat, edge_index, W_ih, W_hh, b_ih, b_hh, emb_weight,
           W_self1, W_neigh1, b1, W_self2, W_neigh2, b2, flat_W, flat_b,
           out_W, out_b):
    f32 = jnp.float32
    bf16 = jnp.bfloat16
    nf = node_feat.astype(bf16)

    epad = ((0, E_PAD - E),)
    src = jnp.pad(edge_index[0], epad, constant_values=N_NODES)
    dst = jnp.pad(edge_index[1], epad)
    zero = jnp.zeros((M_FLAT,), f32)
    m_part = _sc_edge_partials(src, dst, zero)
    m3d = m_part.reshape(2, N_NODES, M_COLS)

    out = pl.pallas_call(
        _fused_body,
        out_shape=jax.ShapeDtypeStruct((N_NODES, 1), f32),
    )(
        nf, flat, m3d,
        W_ih.T.astype(bf16), W_hh.T.astype(bf16),
        (b_ih + b_hh).reshape(1, -1),
        emb_weight, W_self1, W_neigh1, b1.reshape(1, -1),
        W_self2, W_neigh2, b2.reshape(1, -1),
        flat_W, flat_b.reshape(1, -1), out_W, out_b.reshape(1, -1),
    )
    return out[:, 0]


# speedup vs baseline: 1.1670x; 1.0203x over previous
import functools
import jax
import jax.numpy as jnp
from jax import lax
from jax.experimental import pallas as pl
from jax.experimental.pallas import tpu as pltpu
from jax.experimental.pallas import tpu_sc as plsc

N_NODES = 100
FEAT = 32
HIDDEN = 256
EMB = 64
GNN_HID = 256
GNN_OUT = 128
FLAT_DIM = 128
FLAT_OUT = 64
T = 200
E = 3200

N_A = 52
N_B = N_NODES - N_A

N_TILES = 32
E_TILE = 112
E_PAD = N_TILES * E_TILE
M_COLS = 104
M_FLAT = N_NODES * M_COLS


def _sc_edges_body(src_hbm, dst_hbm, zero_hbm, out_hbm,
                   srcv, dstv, valv, fidxv, sh_a, sh_b):
    c = lax.axis_index("c")
    s = lax.axis_index("s")
    wid = s * 2 + c
    base = wid * E_TILE
    pltpu.sync_copy(src_hbm.at[pl.ds(base, E_TILE)], srcv)
    pltpu.sync_copy(dst_hbm.at[pl.ds(base, E_TILE)], dstv)

    @pl.when(jnp.logical_and(c == 0, s == 0))
    def _():
        pltpu.sync_copy(zero_hbm, sh_a)

    @pl.when(jnp.logical_and(c == 1, s == 0))
    def _():
        pltpu.sync_copy(zero_hbm, sh_b)

    for j in range(E_TILE // 16):
        d = dstv[pl.ds(j * 16, 16)]
        sv = srcv[pl.ds(j * 16, 16)]
        fidxv[pl.ds(j * 16, 16)] = d * M_COLS + sv
        valv[pl.ds(j * 16, 16)] = jnp.full((16,), 1.0, jnp.float32)

    plsc.subcore_barrier()

    @pl.when(c == 0)
    def _():
        pltpu.sync_copy(valv, sh_a.at[fidxv], add=True)

    @pl.when(c == 1)
    def _():
        pltpu.sync_copy(valv, sh_b.at[fidxv], add=True)

    plsc.subcore_barrier()

    @pl.when(jnp.logical_and(c == 0, s == 0))
    def _():
        pltpu.sync_copy(sh_a, out_hbm.at[0])

    @pl.when(jnp.logical_and(c == 1, s == 0))
    def _():
        pltpu.sync_copy(sh_b, out_hbm.at[1])


def _sc_edge_partials(src, dst, zero):
    mesh = plsc.VectorSubcoreMesh(core_axis_name="c", subcore_axis_name="s")
    k = functools.partial(
        pl.kernel, mesh=mesh,
        out_type=jax.ShapeDtypeStruct((2, M_FLAT), jnp.float32),
        scratch_types=[
            pltpu.VMEM((E_TILE,), jnp.int32),
            pltpu.VMEM((E_TILE,), jnp.int32),
            pltpu.VMEM((E_TILE,), jnp.float32),
            pltpu.VMEM((E_TILE,), jnp.int32),
            pltpu.VMEM_SHARED((M_FLAT,), jnp.float32),
            pltpu.VMEM_SHARED((M_FLAT,), jnp.float32),
        ],
    )(_sc_edges_body)
    return k(src, dst, zero)


def _fused_body(nf_ref, flat_ref, m_ref, wihT_ref,
                whhT_ref, bias_ref, emb_ref, ws1_ref, wn1_ref, b1_ref,
                ws2_ref, wn2_ref, b2_ref, fw_ref, fb_ref, ow_ref, ob_ref,
                out_ref):
    f32 = jnp.float32
    bf16 = jnp.bfloat16

    wihT = wihT_ref[...]
    whhT = whhT_ref[...]
    bias = bias_ref[...]

    def gates(gi, gh, h):
        r = jnp.tanh((gi[:, :HIDDEN] + gh[:, :HIDDEN]) * 0.5) * 0.5 + 0.5
        z = jnp.tanh((gi[:, HIDDEN:2 * HIDDEN] + gh[:, HIDDEN:2 * HIDDEN]) * 0.5) * 0.5 + 0.5
        n = jnp.tanh(gi[:, 2 * HIDDEN:] + r * gh[:, 2 * HIDDEN:])
        return n + z * (h - n)

    def substep(t, ha, hb):
        x_t = nf_ref[t]
        gia = jnp.dot(x_t[:N_A], wihT, preferred_element_type=f32) + bias
        gha = jnp.dot(ha.astype(bf16), whhT, preferred_element_type=f32)
        gib = jnp.dot(x_t[N_A:], wihT, preferred_element_type=f32) + bias
        ghb = jnp.dot(hb.astype(bf16), whhT, preferred_element_type=f32)
        return gates(gia, gha, ha), gates(gib, ghb, hb)

    def step(i, carry):
        ha, hb = carry
        t = i * 20
        for u in range(20):
            ha, hb = substep(t + u, ha, hb)
        return ha, hb

    ha, hb = jax.lax.fori_loop(
        0, T // 20, step,
        (jnp.zeros((N_A, HIDDEN), f32), jnp.zeros((N_B, HIDDEN), f32)))
    h_last = jnp.concatenate([ha, hb], axis=0)

    m = (m_ref[0] + m_ref[1])[:, :N_NODES]
    cnt = jnp.sum(m, axis=1, keepdims=True)
    inv_cnt = 1.0 / jnp.maximum(cnt, 1.0)

    emb = emb_ref[...]
    gnn_in = jnp.concatenate([h_last, emb], axis=1)
    mean1 = jnp.dot(m, gnn_in, preferred_element_type=f32) * inv_cnt
    h1 = jnp.dot(gnn_in, ws1_ref[...], preferred_element_type=f32)
    h1 = h1 + jnp.dot(mean1, wn1_ref[...], preferred_element_type=f32)
    h1 = jax.nn.relu(h1 + b1_ref[...])

    mean2 = jnp.dot(m, h1, preferred_element_type=f32) * inv_cnt
    h2 = jnp.dot(h1, ws2_ref[...], preferred_element_type=f32)
    h2 = h2 + jnp.dot(mean2, wn2_ref[...], preferred_element_type=f32)
    h2 = h2 + b2_ref[...]

    xflat = jnp.dot(flat_ref[...], fw_ref[...], preferred_element_type=f32) + fb_ref[...]
    xcat = jnp.concatenate([h2, xflat, h_last], axis=1)
    out = jnp.dot(xcat, ow_ref[...], preferred_element_type=f32) + ob_ref[...]
    out_ref[...] = out


def kernel(node_feat, flat, edge_index, W_ih, W_hh, b_ih, b_hh, emb_weight,
           W_self1, W_neigh1, b1, W_self2, W_neigh2, b2, flat_W, flat_b,
           out_W, out_b):
    f32 = jnp.float32
    bf16 = jnp.bfloat16
    nf = node_feat.astype(bf16)

    epad = ((0, E_PAD - E),)
    src = jnp.pad(edge_index[0], epad, constant_values=N_NODES)
    dst = jnp.pad(edge_index[1], epad)
    zero = jnp.zeros((M_FLAT,), f32)
    m_part = _sc_edge_partials(src, dst, zero)
    m3d = m_part.reshape(2, N_NODES, M_COLS)

    out = pl.pallas_call(
        _fused_body,
        out_shape=jax.ShapeDtypeStruct((N_NODES, 1), f32),
    )(
        nf, flat, m3d,
        W_ih.T.astype(bf16), W_hh.T.astype(bf16),
        (b_ih + b_hh).reshape(1, -1),
        emb_weight, W_self1, W_neigh1, b1.reshape(1, -1),
        W_self2, W_neigh2, b2.reshape(1, -1),
        flat_W, flat_b.reshape(1, -1), out_W, out_b.reshape(1, -1),
    )
    return out[:, 0]


# speedup vs baseline: 1.1772x; 1.0088x over previous
import functools
import jax
import jax.numpy as jnp
from jax import lax
from jax.experimental import pallas as pl
from jax.experimental.pallas import tpu as pltpu
from jax.experimental.pallas import tpu_sc as plsc

N_NODES = 100
FEAT = 32
HIDDEN = 256
EMB = 64
GNN_HID = 256
GNN_OUT = 128
FLAT_DIM = 128
FLAT_OUT = 64
T = 200
E = 3200

N_A = 52
N_B = N_NODES - N_A

N_TILES = 32
E_TILE = 112
E_PAD = N_TILES * E_TILE
M_COLS = 104
M_FLAT = N_NODES * M_COLS


def _sc_edges_body(src_hbm, dst_hbm, zero_hbm, out_hbm,
                   srcv, dstv, valv, fidxv, sh_a, sh_b):
    c = lax.axis_index("c")
    s = lax.axis_index("s")
    wid = s * 2 + c
    base = wid * E_TILE
    pltpu.sync_copy(src_hbm.at[pl.ds(base, E_TILE)], srcv)
    pltpu.sync_copy(dst_hbm.at[pl.ds(base, E_TILE)], dstv)

    @pl.when(jnp.logical_and(c == 0, s == 0))
    def _():
        pltpu.sync_copy(zero_hbm, sh_a)

    @pl.when(jnp.logical_and(c == 1, s == 0))
    def _():
        pltpu.sync_copy(zero_hbm, sh_b)

    for j in range(E_TILE // 16):
        d = dstv[pl.ds(j * 16, 16)]
        sv = srcv[pl.ds(j * 16, 16)]
        fidxv[pl.ds(j * 16, 16)] = d * M_COLS + sv
        valv[pl.ds(j * 16, 16)] = jnp.full((16,), 1.0, jnp.float32)

    plsc.subcore_barrier()

    @pl.when(c == 0)
    def _():
        pltpu.sync_copy(valv, sh_a.at[fidxv], add=True)

    @pl.when(c == 1)
    def _():
        pltpu.sync_copy(valv, sh_b.at[fidxv], add=True)

    plsc.subcore_barrier()

    @pl.when(jnp.logical_and(c == 0, s == 0))
    def _():
        pltpu.sync_copy(sh_a, out_hbm.at[0])

    @pl.when(jnp.logical_and(c == 1, s == 0))
    def _():
        pltpu.sync_copy(sh_b, out_hbm.at[1])


def _sc_edge_partials(src, dst, zero):
    mesh = plsc.VectorSubcoreMesh(core_axis_name="c", subcore_axis_name="s")
    k = functools.partial(
        pl.kernel, mesh=mesh,
        out_type=jax.ShapeDtypeStruct((2, M_FLAT), jnp.float32),
        scratch_types=[
            pltpu.VMEM((E_TILE,), jnp.int32),
            pltpu.VMEM((E_TILE,), jnp.int32),
            pltpu.VMEM((E_TILE,), jnp.float32),
            pltpu.VMEM((E_TILE,), jnp.int32),
            pltpu.VMEM_SHARED((M_FLAT,), jnp.float32),
            pltpu.VMEM_SHARED((M_FLAT,), jnp.float32),
        ],
    )(_sc_edges_body)
    return k(src, dst, zero)


def _fused_body(nf_ref, flat_ref, m_ref, wihT_ref,
                whhT_ref, bias_ref, emb_ref, ws1_ref, wn1_ref, b1_ref,
                ws2_ref, wn2_ref, b2_ref, fw_ref, fb_ref, ow_ref, ob_ref,
                out_ref):
    f32 = jnp.float32
    bf16 = jnp.bfloat16

    wihT = wihT_ref[...]
    whhT = whhT_ref[...]
    bias = bias_ref[...]

    def gates(gi, gh, h):
        r = jnp.tanh((gi[:, :HIDDEN] + gh[:, :HIDDEN]) * 0.5) * 0.5 + 0.5
        z = jnp.tanh((gi[:, HIDDEN:2 * HIDDEN] + gh[:, HIDDEN:2 * HIDDEN]) * 0.5) * 0.5 + 0.5
        n = jnp.tanh(gi[:, 2 * HIDDEN:] + r * gh[:, 2 * HIDDEN:])
        return n + z * (h - n)

    def substep(t, ha, hb):
        x_t = nf_ref[t]
        gia = jnp.dot(x_t[:N_A], wihT, preferred_element_type=f32) + bias
        gha = jnp.dot(ha.astype(bf16), whhT, preferred_element_type=f32)
        gib = jnp.dot(x_t[N_A:], wihT, preferred_element_type=f32) + bias
        ghb = jnp.dot(hb.astype(bf16), whhT, preferred_element_type=f32)
        return gates(gia, gha, ha), gates(gib, ghb, hb)

    def step(i, carry):
        ha, hb = carry
        t = i * 40
        for u in range(40):
            ha, hb = substep(t + u, ha, hb)
        return ha, hb

    ha, hb = jax.lax.fori_loop(
        0, T // 40, step,
        (jnp.zeros((N_A, HIDDEN), f32), jnp.zeros((N_B, HIDDEN), f32)))
    h_last = jnp.concatenate([ha, hb], axis=0)

    m = (m_ref[0] + m_ref[1])[:, :N_NODES]
    cnt = jnp.sum(m, axis=1, keepdims=True)
    inv_cnt = 1.0 / jnp.maximum(cnt, 1.0)

    emb = emb_ref[...]
    gnn_in = jnp.concatenate([h_last, emb], axis=1)
    mean1 = jnp.dot(m, gnn_in, preferred_element_type=f32) * inv_cnt
    h1 = jnp.dot(gnn_in, ws1_ref[...], preferred_element_type=f32)
    h1 = h1 + jnp.dot(mean1, wn1_ref[...], preferred_element_type=f32)
    h1 = jax.nn.relu(h1 + b1_ref[...])

    mean2 = jnp.dot(m, h1, preferred_element_type=f32) * inv_cnt
    h2 = jnp.dot(h1, ws2_ref[...], preferred_element_type=f32)
    h2 = h2 + jnp.dot(mean2, wn2_ref[...], preferred_element_type=f32)
    h2 = h2 + b2_ref[...]

    xflat = jnp.dot(flat_ref[...], fw_ref[...], preferred_element_type=f32) + fb_ref[...]
    xcat = jnp.concatenate([h2, xflat, h_last], axis=1)
    out = jnp.dot(xcat, ow_ref[...], preferred_element_type=f32) + ob_ref[...]
    out_ref[...] = out


def kernel(node_feat, flat, edge_index, W_ih, W_hh, b_ih, b_hh, emb_weight,
           W_self1, W_neigh1, b1, W_self2, W_neigh2, b2, flat_W, flat_b,
           out_W, out_b):
    f32 = jnp.float32
    bf16 = jnp.bfloat16
    nf = node_feat.astype(bf16)

    epad = ((0, E_PAD - E),)
    src = jnp.pad(edge_index[0], epad, constant_values=N_NODES)
    dst = jnp.pad(edge_index[1], epad)
    zero = jnp.zeros((M_FLAT,), f32)
    m_part = _sc_edge_partials(src, dst, zero)
    m3d = m_part.reshape(2, N_NODES, M_COLS)

    out = pl.pallas_call(
        _fused_body,
        out_shape=jax.ShapeDtypeStruct((N_NODES, 1), f32),
    )(
        nf, flat, m3d,
        W_ih.T.astype(bf16), W_hh.T.astype(bf16),
        (b_ih + b_hh).reshape(1, -1),
        emb_weight, W_self1, W_neigh1, b1.reshape(1, -1),
        W_self2, W_neigh2, b2.reshape(1, -1),
        flat_W, flat_b.reshape(1, -1), out_W, out_b.reshape(1, -1),
    )
    return out[:, 0]
